# baseline (speedup 1.0000x reference)
import functools

import jax
import jax.numpy as jnp
from jax import lax
from jax.experimental import pallas as pl
from jax.experimental.pallas import tpu as pltpu
from jax.experimental.pallas import tpu_sc as plsc

_NBT = 32
_NAT = 119
_NAP = 16
_E = 64
_NB = 50000
_NU = 400000
_NW = 32
_SUB = 128
_FR = _NAT * _NAP * _NBT
_SPAN = _NU // _NW
_CHUNK = 250
_GCHUNK = 256
_CPW = _SPAN // _CHUNK
_LOAD = 12512
_NPAD = 400064
_ORPW = _SPAN // 2
_ORC = _CHUNK // 2
_AB = 7
_BG = _NAT // _AB
_BR = _AB * _NAP * _NBT


def _build_body(atab_ref, ptab_ref, btab_ref, outl_ref, outr_ref):
    i = pl.program_id(0)
    a = atab_ref[pl.ds(i * _AB, _AB)]
    p = ptab_ref[...]
    b = btab_ref[...]
    ap = a[:, None, :] + p[None, :, :]
    apb = ap[:, :, None, :] + b[None, None, :, :]
    rows = apb.reshape(_BR, _E)
    z = jnp.zeros((_BR, _E), jnp.float32)
    outl_ref[...] = jnp.concatenate([rows, z], axis=1)
    outr_ref[...] = jnp.concatenate([z, rows], axis=1)


_build_fused = pl.pallas_call(
    _build_body,
    grid=(_BG,),
    in_specs=[
        pl.BlockSpec((_NAT, _E), lambda i: (0, 0)),
        pl.BlockSpec((_NAP, _E), lambda i: (0, 0)),
        pl.BlockSpec((_NBT, _E), lambda i: (0, 0)),
    ],
    out_specs=[
        pl.BlockSpec((_BR, 2 * _E), lambda i: (i, 0)),
        pl.BlockSpec((_BR, 2 * _E), lambda i: (i, 0)),
    ],
    out_shape=[
        jax.ShapeDtypeStruct((_FR, 2 * _E), jnp.float32),
        jax.ShapeDtypeStruct((_FR, 2 * _E), jnp.float32),
    ],
)


def _sc_body(
    fusedl, fusedr, btab, ai, pi, bi, out,
    b_v, a_v, p_v, i_v, idx0, idx1, acc0, acc1, s_in, g0, g1, o0, o1,
):
    wid = lax.axis_index("s") * 2 + lax.axis_index("c")
    off = (wid % 2) * 4
    base = pl.multiple_of(wid * _SPAN - off, 8)
    pltpu.sync_copy(btab, b_v)
    for src, dst in ((ai, a_v), (pi, p_v), (bi, i_v)):
        pltpu.async_copy(src.at[pl.ds(base, _LOAD)], dst, s_in)
    for src, dst in ((ai, a_v), (pi, p_v), (bi, i_v)):
        pltpu.make_async_copy(src.at[pl.ds(base, _LOAD)], dst, s_in).wait()

    lanes2 = lax.iota(jnp.int32, 16) * 2

    def compute_idx(jj, idx2):
        for par in range(2):
            for i in range(_SUB // 16):
                s = off + jj * _CHUNK + 32 * i + par + lanes2
                a = plsc.load_gather(a_v, [s])
                p = plsc.load_gather(p_v, [s])
                ib = plsc.load_gather(i_v, [s])
                bt = plsc.load_gather(b_v, [ib])
                idx2[par, pl.ds(i * 16, 16)] = a * 512 + p * 32 + bt

    def start_even(idx2, acc, sem):
        pltpu.async_copy(fusedl.at[idx2.at[0]], acc, sem)

    def wait_even(idx2, acc, sem):
        pltpu.make_async_copy(fusedl.at[idx2.at[0]], acc, sem).wait()

    def start_odd_add(idx2, acc, sem):
        pltpu.async_copy(fusedr.at[idx2.at[1]], acc, sem, add=True)

    def wait_odd(idx2, acc, sem):
        pltpu.make_async_copy(fusedr.at[idx2.at[1]], acc, sem).wait()

    def start_out(c, acc, sem):
        pltpu.async_copy(
            acc.at[pl.ds(0, _ORC)],
            out.at[pl.ds(wid * _ORPW + c * _ORC, _ORC)],
            sem,
        )

    def drain_out(c, acc, sem):
        pltpu.make_async_copy(
            acc.at[pl.ds(0, _ORC)],
            out.at[pl.ds(wid * _ORPW + c * _ORC, _ORC)],
            sem,
        ).wait()

    def pair(t, carry):
        c0 = 2 * t
        c1 = c0 + 1
        compute_idx(c0, idx0)

        @pl.when(t > 0)
        def _():
            drain_out(c0 - 2, acc0, o0)

        start_even(idx0, acc0, g0)
        compute_idx(c1, idx1)
        wait_even(idx0, acc0, g0)
        start_odd_add(idx0, acc0, g0)

        @pl.when(t > 0)
        def _():
            drain_out(c1 - 2, acc1, o1)

        start_even(idx1, acc1, g1)
        wait_odd(idx0, acc0, g0)
        start_out(c0, acc0, o0)
        wait_even(idx1, acc1, g1)
        start_odd_add(idx1, acc1, g1)
        wait_odd(idx1, acc1, g1)
        start_out(c1, acc1, o1)
        return carry

    lax.fori_loop(0, _CPW // 2, pair, 0)
    drain_out(_CPW - 2, acc0, o0)
    drain_out(_CPW - 1, acc1, o1)


_sc_gather = functools.partial(
    pl.kernel,
    out_type=jax.ShapeDtypeStruct((_NU // 2, 2 * _E), jnp.float32),
    mesh=plsc.VectorSubcoreMesh(core_axis_name="c", subcore_axis_name="s"),
    scratch_types=[
        pltpu.VMEM((_NB,), jnp.int32),
        pltpu.VMEM((_LOAD,), jnp.int32),
        pltpu.VMEM((_LOAD,), jnp.int32),
        pltpu.VMEM((_LOAD,), jnp.int32),
        pltpu.VMEM((2, _SUB), jnp.int32),
        pltpu.VMEM((2, _SUB), jnp.int32),
        pltpu.VMEM((_SUB, 2 * _E), jnp.float32),
        pltpu.VMEM((_SUB, 2 * _E), jnp.float32),
        pltpu.SemaphoreType.DMA,
        pltpu.SemaphoreType.DMA,
        pltpu.SemaphoreType.DMA,
        pltpu.SemaphoreType.DMA,
        pltpu.SemaphoreType.DMA,
    ],
    compiler_params=pltpu.CompilerParams(
        needs_layout_passes=False, use_tc_tiling_on_sc=False
    ),
)(_sc_body)


def kernel(B, A, atom_positions, block_id, block_table, atom_table, pos_table):
    b32 = B.astype(jnp.int32)
    pad = _NPAD - _NU
    a32 = jnp.pad(A.astype(jnp.int32), (0, pad))
    p32 = jnp.pad(atom_positions.astype(jnp.int32), (0, pad))
    i32 = jnp.pad(block_id.astype(jnp.int32), (0, pad))
    fusedl, fusedr = _build_fused(atom_table, pos_table, block_table)
    out = _sc_gather(fusedl, fusedr, b32, a32, p32, i32)
    return out.reshape(_NU, _E)
